# baseline (speedup 1.0000x reference)
import functools

import jax
import jax.numpy as jnp
from jax import lax
from jax.experimental import pallas as pl
from jax.experimental.pallas import tpu as pltpu
from jax.experimental.pallas import tpu_sc as plsc

_N = 16384
_C = 1000
_NC = 2
_NS = 16
_NW = _NC * _NS
_BW = _N // _NW
_NCH = _BW // 128

_mesh = plsc.VectorSubcoreMesh(core_axis_name="c", subcore_axis_name="s")


@functools.partial(
    pl.kernel,
    mesh=_mesh,
    out_type=jax.ShapeDtypeStruct((_NW, _NCH, 128), jnp.float32),
    scratch_types=[
        pltpu.VMEM((_BW,), jnp.int32),
        pltpu.VMEM((_NCH, 128), jnp.int32),
        pltpu.VMEM((_NCH, 128), jnp.float32),
        pltpu.SemaphoreType.DMA,
    ],
)
def _sc_gather(logits_hbm, y_hbm, out_hbm, y_v, idx_v, val_v, sem):
    wid = lax.axis_index("s") * _NC + lax.axis_index("c")
    base = wid * _BW
    pltpu.sync_copy(y_hbm.at[pl.ds(base, _BW)], y_v)
    for j in range(_NCH):
        for t in range(8):
            s = j * 128 + t * 16
            rows = base + s + lax.iota(jnp.int32, 16)
            idx_v[j, pl.ds(t * 16, 16)] = rows * _C + y_v[pl.ds(s, 16)]
    copies = [
        pltpu.async_copy(logits_hbm.at[idx_v.at[j]], val_v.at[j], sem)
        for j in range(_NCH)
    ]
    for cp in copies:
        cp.wait()
    pltpu.sync_copy(val_v, out_hbm.at[wid])


def kernel(logits, y):
    out = _sc_gather(logits.reshape(_N * _C), y.astype(jnp.int32))
    return out.reshape(_N)


# speedup vs baseline: 1.5406x; 1.5406x over previous
import jax
import jax.numpy as jnp
from jax import lax
from jax.experimental import pallas as pl

_N = 16384
_C = 1000
_R = 512
_NB = _N // _R


def _body(y_ref, x_ref, o_ref):
    y = y_ref[0, 0, :]
    x = x_ref[...]
    cols = lax.broadcasted_iota(jnp.int32, (_R, _C), 1)
    sel = jnp.where(cols == y[:, None], x, 0.0)
    o_ref[0, 0, :] = jnp.sum(sel, axis=1)


def kernel(logits, y):
    y2 = y.astype(jnp.int32).reshape(_NB, 1, _R)
    out = pl.pallas_call(
        _body,
        grid=(_NB,),
        in_specs=[
            pl.BlockSpec((1, 1, _R), lambda i: (i, 0, 0)),
            pl.BlockSpec((_R, _C), lambda i: (i, 0)),
        ],
        out_specs=pl.BlockSpec((1, 1, _R), lambda i: (i, 0, 0)),
        out_shape=jax.ShapeDtypeStruct((_NB, 1, _R), jnp.float32),
    )(y2, logits)
    return out.reshape(-1)


# speedup vs baseline: 6.5227x; 4.2339x over previous
import jax
import jax.numpy as jnp
from jax import lax
from jax.experimental import pallas as pl

_N = 16384
_C = 1000
_CB = 2048
_NB = _N // _CB


def _body(y_ref, x_ref, o_ref):
    y = y_ref[0, 0, :]
    x = x_ref[...]
    rows = lax.broadcasted_iota(jnp.int32, (_C, _CB), 0)
    sel = jnp.where(rows == y[None, :], x, 0.0)
    o_ref[0, 0, :] = jnp.sum(sel, axis=0)


def kernel(logits, y):
    lt = logits.T
    y2 = y.astype(jnp.int32).reshape(_NB, 1, _CB)
    out = pl.pallas_call(
        _body,
        grid=(_NB,),
        in_specs=[
            pl.BlockSpec((1, 1, _CB), lambda i: (i, 0, 0)),
            pl.BlockSpec((_C, _CB), lambda i: (0, i)),
        ],
        out_specs=pl.BlockSpec((1, 1, _CB), lambda i: (i, 0, 0)),
        out_shape=jax.ShapeDtypeStruct((_NB, 1, _CB), jnp.float32),
    )(y2, lt)
    return out.reshape(-1)
